# Initial kernel scaffold; baseline (speedup 1.0000x reference)
import jax
import jax.numpy as jnp
from jax.experimental import pallas as pl


def kernel(boxes, gauss, pe0, pe1, rep0, rep1, rep2, pos_embed):
    raise NotImplementedError("write your pallas kernel here")


# speedup vs baseline: 1.0746x; 1.0746x over previous
import functools

import jax
import jax.numpy as jnp
import numpy as np
from jax.experimental import pallas as pl
from jax.experimental.pallas import tpu as pltpu

_BB = 16


def _fused_body(boxes_ref, amat_ref, addtab_ref, head_ref, posfull_ref,
                tok_ref, pos_ref):
    bb = boxes_ref.shape[0]
    x = boxes_ref[...].reshape(bb * 400, 2)
    amat = amat_ref[...].astype(jnp.bfloat16).astype(jnp.float32)
    s0 = (2.0 * x[:, 0:1] - 1.0).astype(jnp.bfloat16).astype(jnp.float32)
    s1 = (2.0 * x[:, 1:2] - 1.0).astype(jnp.bfloat16).astype(jnp.float32)
    ang = (2.0 * np.pi) * (s0 * amat[0:1, :] + s1 * amat[1:2, :])
    body = jnp.concatenate([jnp.sin(ang), jnp.cos(ang)], axis=-1)
    body = body.reshape(bb, 400, 128) + addtab_ref[...][None]
    head = jnp.broadcast_to(head_ref[...][None], (bb, 3, 128))
    tok_ref[...] = jnp.concatenate([head, body], axis=1)
    pos_ref[...] = jnp.broadcast_to(posfull_ref[...][None], (bb, 403, 128))


def kernel(boxes, gauss, pe0, pe1, rep0, rep1, rep2, pos_embed):
    bs, t, _ = boxes.shape
    d = pe0.shape[-1]
    r = 2 * t

    boxes2 = boxes.reshape(bs, r, 2)
    pos = pos_embed[0]
    pe_pair = jnp.stack([pe0[0], pe1[0]], axis=0)
    addtab = (pos[:, None, :] + pe_pair[None, :, :]).reshape(r, d)
    head = jnp.concatenate([rep0, rep1 + pe0, rep2 + pe1], axis=0)
    posfull = jnp.concatenate(
        [jnp.zeros((3, d), jnp.float32),
         jnp.broadcast_to(pos[:, None, :], (t, 2, d)).reshape(r, d)], axis=0)

    grid = (bs // _BB,)
    out_shape = (
        jax.ShapeDtypeStruct((bs, 3 + r, d), jnp.float32),
        jax.ShapeDtypeStruct((bs, 3 + r, d), jnp.float32),
    )
    tok, post = pl.pallas_call(
        _fused_body,
        grid=grid,
        in_specs=[
            pl.BlockSpec((_BB, r, 2), lambda i: (i, 0, 0)),
            pl.BlockSpec((2, d // 2), lambda i: (0, 0)),
            pl.BlockSpec((r, d), lambda i: (0, 0)),
            pl.BlockSpec((3, d), lambda i: (0, 0)),
            pl.BlockSpec((3 + r, d), lambda i: (0, 0)),
        ],
        out_specs=(
            pl.BlockSpec((_BB, 3 + r, d), lambda i: (i, 0, 0)),
            pl.BlockSpec((_BB, 3 + r, d), lambda i: (i, 0, 0)),
        ),
        out_shape=out_shape,
    )(boxes2, gauss, addtab, head, posfull)
    return tok, post


# speedup vs baseline: 2.1960x; 2.0435x over previous
import functools

import jax
import jax.numpy as jnp
import numpy as np
from jax.experimental import pallas as pl
from jax.experimental.pallas import tpu as pltpu

_BB = 16


_COS_COEF = (0.9999999922907279, -19.739205554159433, 64.93917223865739,
             -85.45116591186135, 60.17623138974044, -26.000532119652576,
             6.575618022394011)


def _fused_body(boxes_ref, amat_ref, addtab_ref, head_ref, posfull_ref,
                tok_ref, pos_ref):
    bb = boxes_ref.shape[0]
    x = boxes_ref[...].reshape(bb * 400, 2)
    amat = amat_ref[...].astype(jnp.bfloat16).astype(jnp.float32)
    a2 = jnp.concatenate([amat, amat], axis=-1)
    s0 = (2.0 * x[:, 0:1] - 1.0).astype(jnp.bfloat16).astype(jnp.float32)
    s1 = (2.0 * x[:, 1:2] - 1.0).astype(jnp.bfloat16).astype(jnp.float32)
    p = s0 * a2[0:1, :] + s1 * a2[1:2, :]
    shift = jnp.concatenate(
        [jnp.full((1, 64), 0.25, jnp.float32), jnp.zeros((1, 64), jnp.float32)],
        axis=-1)
    u = p - shift
    r = u - jnp.round(u)
    r2 = r * r
    poly = jnp.float32(_COS_COEF[-1])
    for c in _COS_COEF[-2::-1]:
        poly = poly * r2 + jnp.float32(c)
    body = poly.reshape(bb, 400, 128) + addtab_ref[...][None]
    head = jnp.broadcast_to(head_ref[...][None], (bb, 3, 128))
    tok_ref[...] = jnp.concatenate([head, body], axis=1)
    pos_ref[...] = jnp.broadcast_to(posfull_ref[...][None], (bb, 403, 128))


def kernel(boxes, gauss, pe0, pe1, rep0, rep1, rep2, pos_embed):
    bs, t, _ = boxes.shape
    d = pe0.shape[-1]
    r = 2 * t

    boxes2 = boxes.reshape(bs, r, 2)
    pos = pos_embed[0]
    pe_pair = jnp.stack([pe0[0], pe1[0]], axis=0)
    addtab = (pos[:, None, :] + pe_pair[None, :, :]).reshape(r, d)
    head = jnp.concatenate([rep0, rep1 + pe0, rep2 + pe1], axis=0)
    posfull = jnp.concatenate(
        [jnp.zeros((3, d), jnp.float32),
         jnp.broadcast_to(pos[:, None, :], (t, 2, d)).reshape(r, d)], axis=0)

    grid = (bs // _BB,)
    out_shape = (
        jax.ShapeDtypeStruct((bs, 3 + r, d), jnp.float32),
        jax.ShapeDtypeStruct((bs, 3 + r, d), jnp.float32),
    )
    tok, post = pl.pallas_call(
        _fused_body,
        grid=grid,
        in_specs=[
            pl.BlockSpec((_BB, r, 2), lambda i: (i, 0, 0)),
            pl.BlockSpec((2, d // 2), lambda i: (0, 0)),
            pl.BlockSpec((r, d), lambda i: (0, 0)),
            pl.BlockSpec((3, d), lambda i: (0, 0)),
            pl.BlockSpec((3 + r, d), lambda i: (0, 0)),
        ],
        out_specs=(
            pl.BlockSpec((_BB, 3 + r, d), lambda i: (i, 0, 0)),
            pl.BlockSpec((_BB, 3 + r, d), lambda i: (i, 0, 0)),
        ),
        out_shape=out_shape,
    )(boxes2, gauss, addtab, head, posfull)
    return tok, post


# speedup vs baseline: 2.5859x; 1.1776x over previous
import jax
import jax.numpy as jnp
import numpy as np
from jax.experimental import pallas as pl
from jax.experimental.pallas import tpu as pltpu

_BB = 32

_COS_COEF = (0.9999994437335133, -19.73903440290008, 64.93061469583063,
             -85.2959897351225, 58.912659471971246, -21.28321865388442)


def _fused_body(sq_ref, amat_ref, addtab_ref, head_ref, posfull_ref,
                tok_ref, pos_ref):
    bb = sq_ref.shape[2]
    sq = sq_ref[0]
    sb = (2.0 * sq - 1.0).astype(jnp.bfloat16).astype(jnp.float32)
    amat = amat_ref[...].astype(jnp.bfloat16).astype(jnp.float32)
    a2 = jnp.concatenate([amat, amat], axis=-1)
    s0 = sb[0:400, :].T[:, :, None]
    s1 = sb[400:800, :].T[:, :, None]
    p = s0 * a2[0:1, :] + s1 * a2[1:2, :]
    shift = jnp.concatenate(
        [jnp.full((1, 64), 0.25, jnp.float32), jnp.zeros((1, 64), jnp.float32)],
        axis=-1)
    u = p - shift
    r = u - jnp.round(u)
    r2 = r * r
    poly = jnp.float32(_COS_COEF[-1])
    for c in _COS_COEF[-2::-1]:
        poly = poly * r2 + jnp.float32(c)
    body = poly + addtab_ref[...][None]
    head = jnp.broadcast_to(head_ref[...][None], (bb, 3, 128))
    tok_ref[...] = jnp.concatenate([head, body], axis=1)
    pos_ref[...] = jnp.broadcast_to(posfull_ref[...][None], (bb, 403, 128))


def kernel(boxes, gauss, pe0, pe1, rep0, rep1, rep2, pos_embed):
    bs, t, _ = boxes.shape
    d = pe0.shape[-1]
    r = 2 * t
    nsteps = bs // _BB

    x4 = boxes.reshape(bs, r, 2)
    sq = jnp.concatenate([x4[:, :, 0], x4[:, :, 1]], axis=1)
    sq3 = sq.T.reshape(2 * r, nsteps, _BB).transpose(1, 0, 2)

    pos = pos_embed[0]
    pe_pair = jnp.stack([pe0[0], pe1[0]], axis=0)
    addtab = (pos[:, None, :] + pe_pair[None, :, :]).reshape(r, d)
    head = jnp.concatenate([rep0, rep1 + pe0, rep2 + pe1], axis=0)
    posfull = jnp.concatenate(
        [jnp.zeros((3, d), jnp.float32),
         jnp.broadcast_to(pos[:, None, :], (t, 2, d)).reshape(r, d)], axis=0)

    out_shape = (
        jax.ShapeDtypeStruct((bs, 3 + r, d), jnp.float32),
        jax.ShapeDtypeStruct((bs, 3 + r, d), jnp.float32),
    )
    tok, post = pl.pallas_call(
        _fused_body,
        grid=(nsteps,),
        in_specs=[
            pl.BlockSpec((1, 2 * r, _BB), lambda i: (i, 0, 0)),
            pl.BlockSpec((2, d // 2), lambda i: (0, 0)),
            pl.BlockSpec((r, d), lambda i: (0, 0)),
            pl.BlockSpec((3, d), lambda i: (0, 0)),
            pl.BlockSpec((3 + r, d), lambda i: (0, 0)),
        ],
        out_specs=(
            pl.BlockSpec((_BB, 3 + r, d), lambda i: (i, 0, 0)),
            pl.BlockSpec((_BB, 3 + r, d), lambda i: (i, 0, 0)),
        ),
        out_shape=out_shape,
    )(sq3, gauss, addtab, head, posfull)
    return tok, post
